# baseline (speedup 1.0000x reference)
import jax
import jax.numpy as jnp
from jax import lax
from jax.experimental import pallas as pl
from jax.experimental.pallas import tpu as pltpu
from jax.experimental.pallas import tpu_sc as plsc

_B = 16384
_NB = 1024
_E = 64
_SEQ = 50
_NC = 2
_NS = 16
_L = 16
_NW = _NC * _NS
_BPW = _B // _NW
_PB = 2048
_IC = 128


def _sc_body(x_hbm, bins_hbm, table_hbm, out_hbm,
             x_v, bins_v, idx_v, rows_v, gsem, osem):
    wid = lax.axis_index("s") * _NC + lax.axis_index("c")
    base = wid * _BPW
    pltpu.sync_copy(x_hbm.at[pl.ds(base, _BPW)], x_v)
    pltpu.sync_copy(bins_hbm, bins_v)

    def search_chunk(i, carry):
        xv = x_v[pl.ds(i * _L, _L)]
        pos = jnp.zeros((_L,), jnp.int32)
        for bit in (1024, 512, 256, 128, 64, 32, 16, 8, 4, 2, 1):
            cand = pos + bit
            b = plsc.load_gather(bins_v, [cand - 1])
            pos = jnp.where(b < xv, cand, pos)
        idx_v[i // 8, pl.ds((i % 8) * _L, _L)] = pos
        return carry

    lax.fori_loop(0, _BPW // _L, search_chunk, 0)

    for c in range(_BPW // _IC):
        pltpu.async_copy(table_hbm.at[idx_v.at[c]],
                         rows_v.at[pl.ds(c * _IC, _IC)], gsem)
    for c in range(_BPW // _IC):
        pltpu.make_async_copy(table_hbm.at[idx_v.at[c]],
                              rows_v.at[pl.ds(c * _IC, _IC)], gsem).wait()

    def relu_row(r, carry):
        for c in range(_E // _L):
            v = rows_v[r, pl.ds(c * _L, _L)]
            rows_v[r, pl.ds(c * _L, _L)] = jnp.maximum(v, 0.0)
        return carry

    lax.fori_loop(0, _BPW, relu_row, 0)

    def bcast_start(s, carry):
        pltpu.async_copy(rows_v, out_hbm.at[pl.ds(base, _BPW), s], osem)
        return carry

    lax.fori_loop(0, _SEQ, bcast_start, 0)

    def bcast_wait(s, carry):
        pltpu.make_async_copy(
            rows_v, out_hbm.at[pl.ds(base, _BPW), s], osem).wait()
        return carry

    lax.fori_loop(0, _SEQ, bcast_wait, 0)


def kernel(x, table, bins, input_length):
    del input_length
    bins_p = jnp.concatenate(
        [bins, jnp.full((_PB - (_NB - 1),), jnp.inf, dtype=jnp.float32)])
    mesh = plsc.VectorSubcoreMesh(core_axis_name="c", subcore_axis_name="s")
    f = pl.kernel(
        _sc_body,
        mesh=mesh,
        compiler_params=pltpu.CompilerParams(
            needs_layout_passes=False, use_tc_tiling_on_sc=False),
        out_type=jax.ShapeDtypeStruct((_B, _SEQ, _E), jnp.float32),
        scratch_types=[
            pltpu.VMEM((_BPW,), jnp.float32),
            pltpu.VMEM((_PB,), jnp.float32),
            pltpu.VMEM((_BPW // _IC, _IC), jnp.int32),
            pltpu.VMEM((_BPW, _E), jnp.float32),
            pltpu.SemaphoreType.DMA,
            pltpu.SemaphoreType.DMA,
        ],
    )
    return f(x, bins_p, table)


# speedup vs baseline: 1.2666x; 1.2666x over previous
import jax
import jax.numpy as jnp
from jax import lax
from jax.experimental import pallas as pl
from jax.experimental.pallas import tpu as pltpu
from jax.experimental.pallas import tpu_sc as plsc

_B = 16384
_NB = 1024
_E = 64
_EP = 128
_SEQ = 50
_NC = 2
_NS = 16
_L = 16
_NW = _NC * _NS
_BPW = _B // _NW
_PB = 2048
_IC = 128


def _sc_body(x_hbm, bins_hbm, table_hbm, out_hbm,
             x_v, bins_v, idx_v, raw_v, rows_v, gsem, osem):
    wid = lax.axis_index("s") * _NC + lax.axis_index("c")
    base = wid * _BPW
    pltpu.sync_copy(x_hbm.at[pl.ds(base, _BPW)], x_v)
    pltpu.sync_copy(bins_hbm, bins_v)

    def search_chunk(i, carry):
        xv = x_v[pl.ds(i * _L, _L)]
        pos = jnp.zeros((_L,), jnp.int32)
        for bit in (1024, 512, 256, 128, 64, 32, 16, 8, 4, 2, 1):
            cand = pos + bit
            b = plsc.load_gather(bins_v, [cand - 1])
            pos = jnp.where(b < xv, cand, pos)
        idx_v[i // 8, pl.ds((i % 8) * _L, _L)] = pos
        return carry

    lax.fori_loop(0, _BPW // _L, search_chunk, 0)

    for c in range(_BPW // _IC):
        pltpu.async_copy(table_hbm.at[idx_v.at[c]], raw_v, gsem)
        pltpu.make_async_copy(table_hbm.at[idx_v.at[c]], raw_v, gsem).wait()

        def relu_row(r, carry):
            for k in range(_E // _L):
                v = raw_v[r, pl.ds(k * _L, _L)]
                rows_v[c * _IC + r, pl.ds(k * _L, _L)] = jnp.maximum(v, 0.0)
            return carry

        lax.fori_loop(0, _IC, relu_row, 0)

    def bcast_start(s, carry):
        pltpu.async_copy(rows_v, out_hbm.at[pl.ds(base, _BPW), s], osem)
        return carry

    lax.fori_loop(0, _SEQ, bcast_start, 0)

    def bcast_wait(s, carry):
        pltpu.make_async_copy(
            rows_v, out_hbm.at[pl.ds(base, _BPW), s], osem).wait()
        return carry

    lax.fori_loop(0, _SEQ, bcast_wait, 0)


def kernel(x, table, bins, input_length):
    del input_length
    bins_p = jnp.concatenate(
        [bins, jnp.full((_PB - (_NB - 1),), jnp.inf, dtype=jnp.float32)])
    table_p = jnp.pad(table, ((0, 0), (0, _EP - _E)))
    mesh = plsc.VectorSubcoreMesh(core_axis_name="c", subcore_axis_name="s")
    f = pl.kernel(
        _sc_body,
        mesh=mesh,
        compiler_params=pltpu.CompilerParams(
            needs_layout_passes=False, use_tc_tiling_on_sc=True),
        out_type=jax.ShapeDtypeStruct((_B, _SEQ, _E), jnp.float32),
        scratch_types=[
            pltpu.VMEM((_BPW,), jnp.float32),
            pltpu.VMEM((_PB,), jnp.float32),
            pltpu.VMEM((_BPW // _IC, _IC), jnp.int32),
            pltpu.VMEM((_IC, _EP), jnp.float32),
            pltpu.VMEM((_BPW, _E), jnp.float32),
            pltpu.SemaphoreType.DMA,
            pltpu.SemaphoreType.DMA,
        ],
    )
    return f(x, bins_p, table_p)


# speedup vs baseline: 8.5086x; 6.7179x over previous
import jax
import jax.numpy as jnp
from jax.experimental import pallas as pl

_BATCH = 16384
_NBINS = 1024
_EMBED = 64
_SEQ = 50
_BB = 512


def _tc_body(x_ref, bins_ref, tabt_ref, out_ref):
    xb = x_ref[0, 0, :].reshape(1, _BB)
    bins_col = bins_ref[:, :]
    c = (xb > bins_col).astype(jnp.int32)
    idx = jnp.sum(c, axis=0, keepdims=True)
    j = jax.lax.broadcasted_iota(jnp.int32, (_NBINS, _BB), 0)
    onehot_t = (j == idx).astype(jnp.float32)
    relu_t = jnp.maximum(tabt_ref[:, :], 0.0)
    rows_t = jnp.dot(relu_t, onehot_t, preferred_element_type=jnp.float32)
    out_ref[:, :, :] = jnp.broadcast_to(rows_t[None], (_SEQ, _EMBED, _BB))


def kernel(x, table, bins, input_length):
    del input_length
    grid = _BATCH // _BB
    x3 = x.reshape(grid, 1, _BB)
    bins_p = jnp.concatenate(
        [bins, jnp.full((1,), jnp.inf, dtype=bins.dtype)]
    ).reshape(_NBINS, 1)
    tab_t = table.T
    out = pl.pallas_call(
        _tc_body,
        grid=(grid,),
        in_specs=[
            pl.BlockSpec((1, 1, _BB), lambda i: (i, 0, 0)),
            pl.BlockSpec((_NBINS, 1), lambda i: (0, 0)),
            pl.BlockSpec((_EMBED, _NBINS), lambda i: (0, 0)),
        ],
        out_specs=pl.BlockSpec((_SEQ, _EMBED, _BB), lambda i: (0, 0, i)),
        out_shape=jax.ShapeDtypeStruct((_SEQ, _EMBED, _BATCH), jnp.float32),
    )(x3, bins_p, tab_t)
    return jnp.transpose(out, (2, 0, 1))
